# baseline (speedup 1.0000x reference)
import jax
import jax.numpy as jnp
from jax.experimental import pallas as pl
from jax.experimental.pallas import tpu as pltpu

_T, _H, _W, _D = 32, 64, 64, 128
_HS = 2


def _body(st_ref, sh_ref, sw_ref, t_ref, h_ref, w_ref, o_ref):
    hb = _H // _HS
    th = t_ref[0, 0, :] * st_ref[0] + h_ref[...] * sh_ref[0]
    ws = w_ref[...] * sw_ref[0]
    out = th[:, None, :] + ws[None, :, :]
    o_ref[...] = out.reshape(1, hb * _W, _D)


def kernel(t, h, w, temporal_embed, height_embed, width_embed, scale_t, scale_h, scale_w):
    hb = _H // _HS
    return pl.pallas_call(
        _body,
        grid=(_T, _HS),
        in_specs=[
            pl.BlockSpec(memory_space=pltpu.SMEM),
            pl.BlockSpec(memory_space=pltpu.SMEM),
            pl.BlockSpec(memory_space=pltpu.SMEM),
            pl.BlockSpec((1, 1, _D), lambda i, j: (i, 0, 0)),
            pl.BlockSpec((hb, _D), lambda i, j: (j, 0)),
            pl.BlockSpec((_W, _D), lambda i, j: (0, 0)),
        ],
        out_specs=pl.BlockSpec((1, hb * _W, _D), lambda i, j: (0, i * _HS + j, 0)),
        out_shape=jax.ShapeDtypeStruct((1, _T * _H * _W, _D), jnp.float32),
    )(scale_t, scale_h, scale_w,
      temporal_embed[:_T].reshape(_T, 1, _D), height_embed[:_H], width_embed[:_W])


# speedup vs baseline: 1.7198x; 1.7198x over previous
import jax
import jax.numpy as jnp
from jax.experimental import pallas as pl
from jax.experimental.pallas import tpu as pltpu

_T, _H, _W, _D = 32, 64, 64, 128
_TB = 4


def _body(st_ref, sh_ref, sw_ref, t_ref, h_ref, w_ref, o_ref):
    ts = t_ref[:, 0, :] * st_ref[0]
    hs = h_ref[...] * sh_ref[0]
    ws = w_ref[...] * sw_ref[0]
    th = ts[:, None, :] + hs[None, :, :]
    out = th[:, :, None, :] + ws[None, None, :, :]
    o_ref[...] = out.reshape(1, _TB * _H * _W, _D)


def kernel(t, h, w, temporal_embed, height_embed, width_embed, scale_t, scale_h, scale_w):
    return pl.pallas_call(
        _body,
        grid=(_T // _TB,),
        in_specs=[
            pl.BlockSpec(memory_space=pltpu.SMEM),
            pl.BlockSpec(memory_space=pltpu.SMEM),
            pl.BlockSpec(memory_space=pltpu.SMEM),
            pl.BlockSpec((_TB, 1, _D), lambda i: (i, 0, 0)),
            pl.BlockSpec((_H, _D), lambda i: (0, 0)),
            pl.BlockSpec((_W, _D), lambda i: (0, 0)),
        ],
        out_specs=pl.BlockSpec((1, _TB * _H * _W, _D), lambda i: (0, i, 0)),
        out_shape=jax.ShapeDtypeStruct((1, _T * _H * _W, _D), jnp.float32),
    )(scale_t, scale_h, scale_w,
      temporal_embed[:_T].reshape(_T, 1, _D), height_embed[:_H], width_embed[:_W])
